# initial kernel scaffold (unmeasured)
import jax
import jax.numpy as jnp
from jax import lax
from jax.experimental import pallas as pl
from jax.experimental.pallas import tpu as pltpu

N_DEV = 16
PAD = 8


def kernel(x, w_mat):
    m_per, k = x.shape
    _, n = w_mat.shape
    n_per = n // N_DEV
    rows = m_per + PAD

    def body(x_ref, w_ref, out_ref, aug_ref, recv_ref, send_sem, recv_sems):
        my = lax.axis_index("i")

        barrier = pltpu.get_barrier_semaphore()
        for p in range(N_DEV):
            pl.semaphore_signal(
                barrier, inc=1,
                device_id=(p,), device_id_type=pl.DeviceIdType.MESH,
            )
        pl.semaphore_wait(barrier, N_DEV)

        y = jnp.maximum(
            jnp.dot(x_ref[...], w_ref[...], preferred_element_type=jnp.float32),
            0.0,
        )
        amax = jnp.max(y)
        aug_ref[0:m_per, :] = y
        aug_ref[m_per:rows, :] = jnp.broadcast_to(amax, (PAD, n))

        rdmas = []
        for step in range(1, N_DEV):
            p = (my + step) % N_DEV
            rdma = pltpu.make_async_remote_copy(
                src_ref=aug_ref.at[:, pl.ds(p * n_per, n_per)],
                dst_ref=recv_ref.at[my],
                send_sem=send_sem,
                recv_sem=recv_sems.at[my],
                device_id=(p,),
                device_id_type=pl.DeviceIdType.MESH,
            )
            rdma.start()
            rdmas.append(rdma)

        recv_ref[my, :, :] = aug_ref[:, pl.ds(my * n_per, n_per)]

        for d in range(N_DEV):
            @pl.when(d != my)
            def _():
                wait = pltpu.make_async_remote_copy(
                    src_ref=aug_ref.at[:, pl.ds(0, n_per)],
                    dst_ref=recv_ref.at[d],
                    send_sem=send_sem,
                    recv_sem=recv_sems.at[d],
                    device_id=(my,),
                    device_id_type=pl.DeviceIdType.MESH,
                )
                wait.wait_recv()

        gmax = jnp.max(recv_ref[:, m_per, :])
        scale = gmax / 448.0

        for d in range(N_DEV):
            t = recv_ref[d, 0:m_per, :]
            q = (t / scale).astype(jnp.float8_e4m3fn).astype(jnp.float32)
            out_ref[d * m_per:(d + 1) * m_per, :] = q * scale

        for rdma in rdmas:
            rdma.wait_send()

    return pl.pallas_call(
        body,
        out_shape=jax.ShapeDtypeStruct((m_per * N_DEV, n_per), jnp.float32),
        in_specs=[
            pl.BlockSpec(memory_space=pltpu.VMEM),
            pl.BlockSpec(memory_space=pltpu.VMEM),
        ],
        out_specs=pl.BlockSpec(memory_space=pltpu.VMEM),
        scratch_shapes=[
            pltpu.VMEM((rows, n), jnp.float32),
            pltpu.VMEM((N_DEV, rows, n_per), jnp.float32),
            pltpu.SemaphoreType.DMA,
            pltpu.SemaphoreType.DMA((N_DEV,)),
        ],
        compiler_params=pltpu.CompilerParams(collective_id=0),
    )(x, w_mat)


# baseline (device time: 47185 ns/iter reference)
import jax
import jax.numpy as jnp
from jax import lax
from jax.experimental import pallas as pl
from jax.experimental.pallas import tpu as pltpu

N_DEV = 16
PAD = 8


def kernel(x, w_mat):
    m_per, k = x.shape
    _, n = w_mat.shape
    n_per = n // N_DEV
    rows = m_per + PAD

    def body(x_ref, w_ref, out_ref, aug_ref, recv_ref, send_sem, recv_sems):
        my = lax.axis_index("i")

        barrier = pltpu.get_barrier_semaphore()
        for p in range(N_DEV):
            pl.semaphore_signal(
                barrier, inc=1,
                device_id=(p,), device_id_type=pl.DeviceIdType.MESH,
            )
        pl.semaphore_wait(barrier, N_DEV)

        xb = x_ref[...].astype(jnp.bfloat16)
        wb = w_ref[...].astype(jnp.bfloat16)
        y = jnp.maximum(
            jnp.dot(xb, wb, preferred_element_type=jnp.float32),
            0.0,
        )
        amax = jnp.max(y)
        aug_ref[0:m_per, :] = y
        aug_ref[m_per:rows, :] = jnp.broadcast_to(amax, (PAD, n))

        rdmas = []
        for step in range(1, N_DEV):
            p = (my + step) % N_DEV
            rdma = pltpu.make_async_remote_copy(
                src_ref=aug_ref.at[:, pl.ds(p * n_per, n_per)],
                dst_ref=recv_ref.at[my],
                send_sem=send_sem,
                recv_sem=recv_sems.at[my],
                device_id=(p,),
                device_id_type=pl.DeviceIdType.MESH,
            )
            rdma.start()
            rdmas.append(rdma)

        recv_ref[my, :, :] = aug_ref[:, pl.ds(my * n_per, n_per)]

        for d in range(N_DEV):
            @pl.when(d != my)
            def _():
                wait = pltpu.make_async_remote_copy(
                    src_ref=aug_ref.at[:, pl.ds(0, n_per)],
                    dst_ref=recv_ref.at[d],
                    send_sem=send_sem,
                    recv_sem=recv_sems.at[d],
                    device_id=(my,),
                    device_id_type=pl.DeviceIdType.MESH,
                )
                wait.wait_recv()

        gmax = jnp.max(recv_ref[:, m_per, :])
        scale = gmax / 448.0

        for d in range(N_DEV):
            t = recv_ref[d, 0:m_per, :]
            q = (t / scale).astype(jnp.float8_e4m3fn).astype(jnp.float32)
            out_ref[d * m_per:(d + 1) * m_per, :] = q * scale

        for rdma in rdmas:
            rdma.wait_send()

    return pl.pallas_call(
        body,
        out_shape=jax.ShapeDtypeStruct((m_per * N_DEV, n_per), jnp.float32),
        in_specs=[
            pl.BlockSpec(memory_space=pltpu.VMEM),
            pl.BlockSpec(memory_space=pltpu.VMEM),
        ],
        out_specs=pl.BlockSpec(memory_space=pltpu.VMEM),
        scratch_shapes=[
            pltpu.VMEM((rows, n), jnp.float32),
            pltpu.VMEM((N_DEV, rows, n_per), jnp.float32),
            pltpu.SemaphoreType.DMA,
            pltpu.SemaphoreType.DMA((N_DEV,)),
        ],
        compiler_params=pltpu.CompilerParams(
            collective_id=0,
            vmem_limit_bytes=110 * 1024 * 1024,
        ),
    )(x, w_mat)


# device time: 35474 ns/iter; 1.3301x vs baseline; 1.3301x over previous
import jax
import jax.numpy as jnp
from jax import lax
from jax.experimental import pallas as pl
from jax.experimental.pallas import tpu as pltpu

N_DEV = 16
APAD = 8


def kernel(x, w_mat):
    m_per, k = x.shape
    _, n = w_mat.shape
    n_per = n // N_DEV

    def body(x_ref, w_ref, out_ref, wbuf, ysend, recv_ref,
             amax_send, amax_recv, wsems, tsend_sem, trecv_sems,
             asend_sem, arecv_sems):
        my = lax.axis_index("i")

        barrier = pltpu.get_barrier_semaphore()
        for p in range(N_DEV):
            pl.semaphore_signal(
                barrier, inc=1,
                device_id=(p,), device_id_type=pl.DeviceIdType.MESH,
            )

        orders = list(range(1, N_DEV)) + [0]

        def w_dma(idx):
            p = (my + orders[idx]) % N_DEV
            return pltpu.make_async_copy(
                w_ref.at[:, pl.ds(p * n_per, n_per)],
                wbuf.at[idx % 2],
                wsems.at[idx % 2],
            )

        w_dma(0).start()
        xb = x_ref[...].astype(jnp.bfloat16)

        amax = jnp.float32(0.0)
        tile_rdmas = []
        for idx in range(N_DEV):
            s = orders[idx]
            p = (my + s) % N_DEV
            if idx + 1 < N_DEV:
                w_dma(idx + 1).start()
            w_dma(idx).wait()
            wb = wbuf[idx % 2].astype(jnp.bfloat16)
            y = jnp.maximum(
                jnp.dot(xb, wb, preferred_element_type=jnp.float32), 0.0
            )
            amax = jnp.maximum(amax, jnp.max(y))
            yb = y.astype(jnp.bfloat16)
            if s != 0:
                ysend[:, pl.ds(p * n_per, n_per)] = yb
                if idx == 0:
                    pl.semaphore_wait(barrier, N_DEV)
                rdma = pltpu.make_async_remote_copy(
                    src_ref=ysend.at[:, pl.ds(p * n_per, n_per)],
                    dst_ref=recv_ref.at[my],
                    send_sem=tsend_sem,
                    recv_sem=trecv_sems.at[my],
                    device_id=(p,),
                    device_id_type=pl.DeviceIdType.MESH,
                )
                rdma.start()
                tile_rdmas.append(rdma)
            else:
                recv_ref[my, :, :] = yb

        amax_send[:, :] = jnp.broadcast_to(amax, (APAD, n_per))
        amax_recv[my, :, :] = jnp.broadcast_to(amax, (APAD, n_per))
        amax_rdmas = []
        for s in range(1, N_DEV):
            p = (my + s) % N_DEV
            rdma = pltpu.make_async_remote_copy(
                src_ref=amax_send,
                dst_ref=amax_recv.at[my],
                send_sem=asend_sem,
                recv_sem=arecv_sems.at[my],
                device_id=(p,),
                device_id_type=pl.DeviceIdType.MESH,
            )
            rdma.start()
            amax_rdmas.append(rdma)

        for s in range(1, N_DEV):
            d = (my + s) % N_DEV
            pltpu.make_async_remote_copy(
                src_ref=ysend.at[:, pl.ds(0, n_per)],
                dst_ref=recv_ref.at[d],
                send_sem=tsend_sem,
                recv_sem=trecv_sems.at[d],
                device_id=(my,),
                device_id_type=pl.DeviceIdType.MESH,
            ).wait_recv()
            pltpu.make_async_remote_copy(
                src_ref=amax_send,
                dst_ref=amax_recv.at[d],
                send_sem=asend_sem,
                recv_sem=arecv_sems.at[d],
                device_id=(my,),
                device_id_type=pl.DeviceIdType.MESH,
            ).wait_recv()

        gmax = jnp.max(amax_recv[:, 0, :])
        scale = gmax / 448.0

        for d in range(N_DEV):
            t = recv_ref[d].astype(jnp.float32)
            q = (t / scale).astype(jnp.float8_e4m3fn).astype(jnp.float32)
            out_ref[d * m_per:(d + 1) * m_per, :] = q * scale

        for rdma in tile_rdmas:
            rdma.wait_send()
        for rdma in amax_rdmas:
            rdma.wait_send()

    return pl.pallas_call(
        body,
        out_shape=jax.ShapeDtypeStruct((m_per * N_DEV, n_per), jnp.float32),
        in_specs=[
            pl.BlockSpec(memory_space=pltpu.VMEM),
            pl.BlockSpec(memory_space=pl.ANY),
        ],
        out_specs=pl.BlockSpec(memory_space=pltpu.VMEM),
        scratch_shapes=[
            pltpu.VMEM((2, k, n_per), jnp.float32),
            pltpu.VMEM((m_per, n), jnp.bfloat16),
            pltpu.VMEM((N_DEV, m_per, n_per), jnp.bfloat16),
            pltpu.VMEM((APAD, n_per), jnp.float32),
            pltpu.VMEM((N_DEV, APAD, n_per), jnp.float32),
            pltpu.SemaphoreType.DMA((2,)),
            pltpu.SemaphoreType.DMA,
            pltpu.SemaphoreType.DMA((N_DEV,)),
            pltpu.SemaphoreType.DMA,
            pltpu.SemaphoreType.DMA((N_DEV,)),
        ],
        compiler_params=pltpu.CompilerParams(
            collective_id=0,
            vmem_limit_bytes=64 * 1024 * 1024,
        ),
    )(x, w_mat)


# device time: 31602 ns/iter; 1.4931x vs baseline; 1.1225x over previous
import jax
import jax.numpy as jnp
from jax import lax
from jax.experimental import pallas as pl
from jax.experimental.pallas import tpu as pltpu

N_DEV = 16
NCH = 4
BPC = N_DEV // NCH
APAD = 8


def kernel(x, w_mat):
    m_per, k = x.shape
    _, n = w_mat.shape
    n_per = n // N_DEV
    c_w = n // NCH

    def body(x_ref, w_ref, out_ref, wbuf, ysend, recv_ref,
             am1_send, am1_recv, am2_send, am2_recv,
             wsems, tsend_sem, trecv_sems, am1_sems, am2_sems):
        my = lax.axis_index("i")
        grp = my // BPC
        pos = my % BPC

        barrier = pltpu.get_barrier_semaphore()
        for p in range(N_DEV):
            pl.semaphore_signal(
                barrier, inc=1,
                device_id=(p,), device_id_type=pl.DeviceIdType.MESH,
            )

        def w_dma(c_idx):
            c = (grp + 1 + c_idx) % NCH
            return pltpu.make_async_copy(
                w_ref.at[:, pl.ds(c * c_w, c_w)],
                wbuf.at[c_idx % 2],
                wsems.at[c_idx % 2],
            )

        w_dma(0).start()
        xb = x_ref[...].astype(jnp.bfloat16)

        amax = jnp.float32(0.0)
        for c_idx in range(NCH):
            c = (grp + 1 + c_idx) % NCH
            if c_idx + 1 < NCH:
                w_dma(c_idx + 1).start()
            w_dma(c_idx).wait()
            wb = wbuf[c_idx % 2].astype(jnp.bfloat16)
            y = jnp.maximum(
                jnp.dot(xb, wb, preferred_element_type=jnp.float32), 0.0
            )
            amax = jnp.maximum(amax, jnp.max(y))
            yb = y.astype(jnp.bfloat16)
            ysend[:, pl.ds(c * c_w, c_w)] = yb
            if c_idx == 0:
                pl.semaphore_wait(barrier, N_DEV)
            for b in range(BPC):
                p = c * BPC + b
                @pl.when(p != my)
                def _():
                    pltpu.make_async_remote_copy(
                        src_ref=ysend.at[:, pl.ds(p * n_per, n_per)],
                        dst_ref=recv_ref.at[my],
                        send_sem=tsend_sem,
                        recv_sem=trecv_sems.at[my],
                        device_id=(p,),
                        device_id_type=pl.DeviceIdType.MESH,
                    ).start()

        recv_ref[my, :, :] = ysend[:, pl.ds(my * n_per, n_per)]

        am1_send[:, :] = jnp.broadcast_to(amax, (APAD, n_per))
        am1_rdmas = []
        for j in range(1, BPC):
            partner = grp * BPC + (pos + j) % BPC
            r = pltpu.make_async_remote_copy(
                src_ref=am1_send,
                dst_ref=am1_recv.at[j - 1],
                send_sem=am1_sems.at[j - 1],
                recv_sem=am1_sems.at[BPC - 1 + j - 1],
                device_id=(partner,),
                device_id_type=pl.DeviceIdType.MESH,
            )
            r.start()
            am1_rdmas.append(r)
        for r in am1_rdmas:
            r.wait_recv()
        pmax = jnp.maximum(amax, jnp.max(am1_recv[:, 0, :]))

        am2_send[:, :] = jnp.broadcast_to(pmax, (APAD, n_per))
        am2_rdmas = []
        for j in range(1, NCH):
            partner = ((grp + j) % NCH) * BPC + pos
            r = pltpu.make_async_remote_copy(
                src_ref=am2_send,
                dst_ref=am2_recv.at[j - 1],
                send_sem=am2_sems.at[j - 1],
                recv_sem=am2_sems.at[NCH - 1 + j - 1],
                device_id=(partner,),
                device_id_type=pl.DeviceIdType.MESH,
            )
            r.start()
            am2_rdmas.append(r)
        for r in am2_rdmas:
            r.wait_recv()
        gmax = jnp.maximum(pmax, jnp.max(am2_recv[:, 0, :]))
        scale = gmax / 448.0

        for s in range(1, N_DEV):
            d = (my + s) % N_DEV
            pltpu.make_async_remote_copy(
                src_ref=ysend.at[:, pl.ds(0, n_per)],
                dst_ref=recv_ref.at[d],
                send_sem=tsend_sem,
                recv_sem=trecv_sems.at[d],
                device_id=(my,),
                device_id_type=pl.DeviceIdType.MESH,
            ).wait_recv()
        for d in range(N_DEV):
            t = recv_ref[d].astype(jnp.float32)
            q = (t / scale).astype(jnp.float8_e4m3fn).astype(jnp.float32)
            out_ref[d * m_per:(d + 1) * m_per, :] = q * scale

        for _ in range(N_DEV - 1):
            pltpu.make_async_remote_copy(
                src_ref=ysend.at[:, pl.ds(0, n_per)],
                dst_ref=recv_ref.at[0],
                send_sem=tsend_sem,
                recv_sem=trecv_sems.at[0],
                device_id=(my,),
                device_id_type=pl.DeviceIdType.MESH,
            ).wait_send()
        for r in am1_rdmas:
            r.wait_send()
        for r in am2_rdmas:
            r.wait_send()

    return pl.pallas_call(
        body,
        out_shape=jax.ShapeDtypeStruct((m_per * N_DEV, n_per), jnp.float32),
        in_specs=[
            pl.BlockSpec(memory_space=pltpu.VMEM),
            pl.BlockSpec(memory_space=pl.ANY),
        ],
        out_specs=pl.BlockSpec(memory_space=pltpu.VMEM),
        scratch_shapes=[
            pltpu.VMEM((2, k, c_w), jnp.float32),
            pltpu.VMEM((m_per, n), jnp.bfloat16),
            pltpu.VMEM((N_DEV, m_per, n_per), jnp.bfloat16),
            pltpu.VMEM((APAD, n_per), jnp.float32),
            pltpu.VMEM((BPC - 1, APAD, n_per), jnp.float32),
            pltpu.VMEM((APAD, n_per), jnp.float32),
            pltpu.VMEM((NCH - 1, APAD, n_per), jnp.float32),
            pltpu.SemaphoreType.DMA((2,)),
            pltpu.SemaphoreType.DMA,
            pltpu.SemaphoreType.DMA((N_DEV,)),
            pltpu.SemaphoreType.DMA((2 * (BPC - 1),)),
            pltpu.SemaphoreType.DMA((2 * (NCH - 1),)),
        ],
        compiler_params=pltpu.CompilerParams(
            collective_id=0,
            vmem_limit_bytes=64 * 1024 * 1024,
        ),
    )(x, w_mat)


# device time: 29589 ns/iter; 1.5947x vs baseline; 1.0680x over previous
import jax
import jax.numpy as jnp
from jax import lax
from jax.experimental import pallas as pl
from jax.experimental.pallas import tpu as pltpu

N_DEV = 16
NCH = 4
BPC = N_DEV // NCH
APAD = 8


def kernel(x, w_mat):
    m_per, k = x.shape
    _, n = w_mat.shape
    n_per = n // N_DEV
    c_w = n // NCH

    def body(x_ref, w_ref, out_ref, wbuf, ysend, recv_ref, qstage,
             am1_send, am1_recv, am2_send, am2_recv,
             wsems, tsend_sem, trecv_sems, am1_sems, am2_sems, out_sems):
        my = lax.axis_index("i")
        grp = my // BPC
        pos = my % BPC

        barrier = pltpu.get_barrier_semaphore()
        for p in range(N_DEV):
            pl.semaphore_signal(
                barrier, inc=1,
                device_id=(p,), device_id_type=pl.DeviceIdType.MESH,
            )

        def w_dma(c_idx):
            c = (grp + 1 + c_idx) % NCH
            return pltpu.make_async_copy(
                w_ref.at[:, pl.ds(c * c_w, c_w)],
                wbuf.at[c_idx % 2],
                wsems.at[c_idx % 2],
            )

        def send_tile(p):
            @pl.when(p != my)
            def _():
                pltpu.make_async_remote_copy(
                    src_ref=ysend.at[:, pl.ds(p * n_per, n_per)],
                    dst_ref=recv_ref.at[my],
                    send_sem=tsend_sem,
                    recv_sem=trecv_sems.at[my],
                    device_id=(p,),
                    device_id_type=pl.DeviceIdType.MESH,
                ).start()

        w_dma(0).start()
        xb = x_ref[...].astype(jnp.bfloat16)

        amax = jnp.float32(0.0)
        am1_rdmas = []
        for c_idx in range(NCH):
            c = (grp + 1 + c_idx) % NCH
            if c_idx + 1 < NCH:
                w_dma(c_idx + 1).start()
            w_dma(c_idx).wait()
            wb = wbuf[c_idx % 2].astype(jnp.bfloat16)
            y = jnp.maximum(
                jnp.dot(xb, wb, preferred_element_type=jnp.float32), 0.0
            )
            amax = jnp.maximum(amax, jnp.max(y))
            yb = y.astype(jnp.bfloat16)
            if c_idx == 0:
                pl.semaphore_wait(barrier, N_DEV)
            if c_idx < NCH - 1:
                ysend[:, pl.ds(c * c_w, c_w)] = yb
                for b in range(BPC):
                    send_tile(c * BPC + b)
            else:
                am1_send[:, :] = jnp.broadcast_to(amax, (APAD, n_per))
                for j in range(1, BPC):
                    partner = grp * BPC + (pos + j) % BPC
                    r = pltpu.make_async_remote_copy(
                        src_ref=am1_send,
                        dst_ref=am1_recv.at[j - 1],
                        send_sem=am1_sems.at[j - 1],
                        recv_sem=am1_sems.at[BPC - 1 + j - 1],
                        device_id=(partner,),
                        device_id_type=pl.DeviceIdType.MESH,
                    )
                    r.start()
                    am1_rdmas.append(r)
                ysend[:, pl.ds(c * c_w, c_w)] = yb
                for b in range(BPC):
                    send_tile(c * BPC + b)

        recv_ref[my, :, :] = ysend[:, pl.ds(my * n_per, n_per)]

        for r in am1_rdmas:
            r.wait_recv()
        pmax = jnp.maximum(amax, jnp.max(am1_recv[:, 0, :]))
        am2_send[:, :] = jnp.broadcast_to(pmax, (APAD, n_per))
        am2_rdmas = []
        for j in range(1, NCH):
            partner = ((grp + j) % NCH) * BPC + pos
            r = pltpu.make_async_remote_copy(
                src_ref=am2_send,
                dst_ref=am2_recv.at[j - 1],
                send_sem=am2_sems.at[j - 1],
                recv_sem=am2_sems.at[NCH - 1 + j - 1],
                device_id=(partner,),
                device_id_type=pl.DeviceIdType.MESH,
            )
            r.start()
            am2_rdmas.append(r)
        for r in am2_rdmas:
            r.wait_recv()
        gmax = jnp.maximum(pmax, jnp.max(am2_recv[:, 0, :]))
        scale = gmax / 448.0

        def quant_out(d):
            t = recv_ref[d].astype(jnp.float32)
            q = (t / scale).astype(jnp.float8_e4m3fn).astype(jnp.float32)
            qstage[d, :, :] = q * scale
            return pltpu.make_async_copy(
                qstage.at[d],
                out_ref.at[pl.ds(d * m_per, m_per), :],
                out_sems.at[d],
            )

        out_dmas = [quant_out(my)]
        out_dmas[0].start()
        for s in range(1, N_DEV):
            d = (my + s) % N_DEV
            pltpu.make_async_remote_copy(
                src_ref=ysend.at[:, pl.ds(0, n_per)],
                dst_ref=recv_ref.at[d],
                send_sem=tsend_sem,
                recv_sem=trecv_sems.at[d],
                device_id=(my,),
                device_id_type=pl.DeviceIdType.MESH,
            ).wait_recv()
            dma = quant_out(d)
            dma.start()
            out_dmas.append(dma)
        for dma in out_dmas:
            dma.wait()

        for _ in range(N_DEV - 1):
            pltpu.make_async_remote_copy(
                src_ref=ysend.at[:, pl.ds(0, n_per)],
                dst_ref=recv_ref.at[0],
                send_sem=tsend_sem,
                recv_sem=trecv_sems.at[0],
                device_id=(my,),
                device_id_type=pl.DeviceIdType.MESH,
            ).wait_send()
        for r in am1_rdmas:
            r.wait_send()
        for r in am2_rdmas:
            r.wait_send()

    return pl.pallas_call(
        body,
        out_shape=jax.ShapeDtypeStruct((m_per * N_DEV, n_per), jnp.float32),
        in_specs=[
            pl.BlockSpec(memory_space=pltpu.VMEM),
            pl.BlockSpec(memory_space=pl.ANY),
        ],
        out_specs=pl.BlockSpec(memory_space=pl.ANY),
        scratch_shapes=[
            pltpu.VMEM((2, k, c_w), jnp.float32),
            pltpu.VMEM((m_per, n), jnp.bfloat16),
            pltpu.VMEM((N_DEV, m_per, n_per), jnp.bfloat16),
            pltpu.VMEM((N_DEV, m_per, n_per), jnp.float32),
            pltpu.VMEM((APAD, n_per), jnp.float32),
            pltpu.VMEM((BPC - 1, APAD, n_per), jnp.float32),
            pltpu.VMEM((APAD, n_per), jnp.float32),
            pltpu.VMEM((NCH - 1, APAD, n_per), jnp.float32),
            pltpu.SemaphoreType.DMA((2,)),
            pltpu.SemaphoreType.DMA,
            pltpu.SemaphoreType.DMA((N_DEV,)),
            pltpu.SemaphoreType.DMA((2 * (BPC - 1),)),
            pltpu.SemaphoreType.DMA((2 * (NCH - 1),)),
            pltpu.SemaphoreType.DMA((N_DEV,)),
        ],
        compiler_params=pltpu.CompilerParams(
            collective_id=0,
            vmem_limit_bytes=64 * 1024 * 1024,
        ),
    )(x, w_mat)


# device time: 26700 ns/iter; 1.7672x vs baseline; 1.1082x over previous
import jax
import jax.numpy as jnp
from jax import lax
from jax.experimental import pallas as pl
from jax.experimental.pallas import tpu as pltpu

N_DEV = 16
NCH = 4
BPC = N_DEV // NCH


def kernel(x, w_mat):
    m_per, k = x.shape
    _, n = w_mat.shape
    n_per = n // N_DEV
    c_w = n // NCH

    def body(x_ref, w_ref, out_ref, wbuf, ysend, recv_ref, qstage,
             wsems, tsend_sem, trecv_sems, amax_sems, out_sems):
        my = lax.axis_index("i")
        grp = my // BPC

        barrier = pltpu.get_barrier_semaphore()
        for p in range(N_DEV):
            pl.semaphore_signal(
                barrier, inc=1,
                device_id=(p,), device_id_type=pl.DeviceIdType.MESH,
            )

        def w_dma(c_idx):
            c = (grp + 1 + c_idx) % NCH
            return pltpu.make_async_copy(
                w_ref.at[:, pl.ds(c * c_w, c_w)],
                wbuf.at[c_idx % 2],
                wsems.at[c_idx % 2],
            )

        def send_tile(p):
            @pl.when(p != my)
            def _():
                pltpu.make_async_remote_copy(
                    src_ref=ysend.at[:, pl.ds(p * n_per, n_per)],
                    dst_ref=recv_ref.at[my],
                    send_sem=tsend_sem,
                    recv_sem=trecv_sems.at[my],
                    device_id=(p,),
                    device_id_type=pl.DeviceIdType.MESH,
                ).start()

        w_dma(0).start()
        xb = x_ref[...].astype(jnp.bfloat16)

        amax = jnp.float32(0.0)
        for c_idx in range(NCH):
            c = (grp + 1 + c_idx) % NCH
            if c_idx + 1 < NCH:
                w_dma(c_idx + 1).start()
            w_dma(c_idx).wait()
            wb = wbuf[c_idx % 2].astype(jnp.bfloat16)
            y = jnp.maximum(
                jnp.dot(xb, wb, preferred_element_type=jnp.float32), 0.0
            )
            amax = jnp.maximum(amax, jnp.max(y))
            yb = y.astype(jnp.bfloat16)
            if c_idx == 0:
                pl.semaphore_wait(barrier, N_DEV)
            if c_idx == NCH - 1:
                amax_bits = lax.bitcast_convert_type(amax, jnp.int32)
                for s in range(1, N_DEV):
                    p = (my + s) % N_DEV
                    pl.semaphore_signal(
                        amax_sems.at[my], inc=amax_bits,
                        device_id=(p,),
                        device_id_type=pl.DeviceIdType.MESH,
                    )
            ysend[:, pl.ds(c * c_w, c_w)] = yb
            for b in range(BPC):
                send_tile(c * BPC + b)

        recv_ref[my, :, :] = ysend[:, pl.ds(my * n_per, n_per)]

        gmax = amax
        for s in range(1, N_DEV):
            d = (my + s) % N_DEV
            pl.semaphore_wait(amax_sems.at[d], 1)
            rest = pl.semaphore_read(amax_sems.at[d])
            peer_amax = lax.bitcast_convert_type(rest + 1, jnp.float32)
            gmax = jnp.maximum(gmax, peer_amax)
            pl.semaphore_wait(amax_sems.at[d], rest)
        scale = gmax / 448.0

        def quant_out(d):
            t = recv_ref[d].astype(jnp.float32)
            q = (t / scale).astype(jnp.float8_e4m3fn).astype(jnp.float32)
            qstage[d, :, :] = q * scale
            return pltpu.make_async_copy(
                qstage.at[d],
                out_ref.at[pl.ds(d * m_per, m_per), :],
                out_sems.at[d],
            )

        out_dmas = [quant_out(my)]
        out_dmas[0].start()
        for s in range(1, N_DEV):
            d = (my + s) % N_DEV
            pltpu.make_async_remote_copy(
                src_ref=ysend.at[:, pl.ds(0, n_per)],
                dst_ref=recv_ref.at[d],
                send_sem=tsend_sem,
                recv_sem=trecv_sems.at[d],
                device_id=(my,),
                device_id_type=pl.DeviceIdType.MESH,
            ).wait_recv()
            dma = quant_out(d)
            dma.start()
            out_dmas.append(dma)
        for dma in out_dmas:
            dma.wait()

        for _ in range(N_DEV - 1):
            pltpu.make_async_remote_copy(
                src_ref=ysend.at[:, pl.ds(0, n_per)],
                dst_ref=recv_ref.at[0],
                send_sem=tsend_sem,
                recv_sem=trecv_sems.at[0],
                device_id=(my,),
                device_id_type=pl.DeviceIdType.MESH,
            ).wait_send()

    return pl.pallas_call(
        body,
        out_shape=jax.ShapeDtypeStruct((m_per * N_DEV, n_per), jnp.float32),
        in_specs=[
            pl.BlockSpec(memory_space=pltpu.VMEM),
            pl.BlockSpec(memory_space=pl.ANY),
        ],
        out_specs=pl.BlockSpec(memory_space=pl.ANY),
        scratch_shapes=[
            pltpu.VMEM((2, k, c_w), jnp.float32),
            pltpu.VMEM((m_per, n), jnp.bfloat16),
            pltpu.VMEM((N_DEV, m_per, n_per), jnp.bfloat16),
            pltpu.VMEM((N_DEV, m_per, n_per), jnp.float32),
            pltpu.SemaphoreType.DMA((2,)),
            pltpu.SemaphoreType.DMA,
            pltpu.SemaphoreType.DMA((N_DEV,)),
            pltpu.SemaphoreType.REGULAR((N_DEV,)),
            pltpu.SemaphoreType.DMA((N_DEV,)),
        ],
        compiler_params=pltpu.CompilerParams(
            collective_id=0,
            vmem_limit_bytes=64 * 1024 * 1024,
        ),
    )(x, w_mat)
